# baseline (speedup 1.0000x reference)
import functools

import numpy as np
import jax
import jax.numpy as jnp
from jax import lax
from jax.experimental import pallas as pl
from jax.experimental.pallas import tpu as pltpu
from jax.experimental.pallas import tpu_sc as plsc

_B = 1024
_N = 262144
_NODE_DIM = 20
_X_DIM = 3


def _schedule_consts():
    start = jnp.arccos(jnp.float32(0.95))
    end = jnp.arccos(jnp.float32(0.02))
    angles = start + jnp.float32(1.0) * (end - start)
    alpha = jnp.cos(angles)
    sigma = jnp.sin(angles)
    L = jnp.log(jnp.float32(1.0) / sigma)
    s2 = sigma * sigma
    half = jnp.float32(0.5)
    ch20 = jnp.float32(20.0) * (L - half * (jnp.float32(1.0) - s2))
    ha2 = half * (alpha * alpha)
    return L, s2, ch20, ha2

_GRID1 = 2
_BNL = _N // _GRID1


def _tc_sq_body(h_ref, x_ref, o_ref):
    hv = h_ref[...]
    xv = x_ref[...]
    o_ref[...] = jnp.sum(hv * hv, axis=0) + jnp.sum(xv * xv, axis=0)


def _tc_sq(h, x):
    return pl.pallas_call(
        _tc_sq_body,
        grid=(_GRID1,),
        in_specs=[
            pl.BlockSpec((_NODE_DIM, _BNL), lambda i: (0, i)),
            pl.BlockSpec((_X_DIM, _BNL), lambda i: (0, i)),
        ],
        out_specs=pl.BlockSpec((_BNL,), lambda i: (i,)),
        out_shape=jax.ShapeDtypeStruct((_N,), jnp.float32),
    )(h.T, x.T)


_NW = 32
_CHUNK = _N // _NW
_NVEC = _CHUNK // 16


@functools.cache
def _get_sc_segsum():
    mesh = plsc.VectorSubcoreMesh(core_axis_name="c", subcore_axis_name="s")

    @functools.partial(
        pl.kernel,
        mesh=mesh,
        out_type=[
            jax.ShapeDtypeStruct((_NW, _B), jnp.float32),
            jax.ShapeDtypeStruct((_NW, _B), jnp.float32),
        ],
        scratch_types=[
            pltpu.VMEM((_CHUNK,), jnp.float32),
            pltpu.VMEM((_CHUNK,), jnp.int32),
            pltpu.VMEM((16 * _B,), jnp.float32),
            pltpu.VMEM((16 * _B,), jnp.float32),
            pltpu.VMEM((_B,), jnp.float32),
            pltpu.VMEM((_B,), jnp.float32),
        ],
        compiler_params=pltpu.CompilerParams(needs_layout_passes=False),
    )
    def _sc_segsum(sq_hbm, idx_hbm, psum_hbm, pcnt_hbm,
                   sq_v, idx_v, acc_s, acc_c, red_s, red_c):
        wid = lax.axis_index("s") * 2 + lax.axis_index("c")
        base = wid * _CHUNK
        pltpu.sync_copy(sq_hbm.at[pl.ds(base, _CHUNK)], sq_v)
        pltpu.sync_copy(idx_hbm.at[pl.ds(base, _CHUNK)], idx_v)

        zero = jnp.zeros((16,), jnp.float32)
        ones = jnp.ones((16,), jnp.float32)
        lane_off = lax.iota(jnp.int32, 16) * _B

        lo = lax.reduce_min(idx_v[pl.ds(0, 16)], axes=(0,))
        hi = lax.reduce_max(idx_v[pl.ds(_CHUNK - 16, 16)], axes=(0,))
        c0 = lo // 16
        c1 = hi // 16 + 1

        @plsc.parallel_loop(c0, c1, 1, unroll=2)
        def _zero_body(c):
            for r in range(16):
                acc_s[pl.ds(r * _B + c * 16, 16)] = zero
                acc_c[pl.ds(r * _B + c * 16, 16)] = zero

        @plsc.parallel_loop(0, _B // 16, 1, unroll=8)
        def _zero_red_body(c):
            red_s[pl.ds(c * 16, 16)] = zero
            red_c[pl.ds(c * 16, 16)] = zero

        @plsc.parallel_loop(0, _NVEC, 1, unroll=8)
        def _acc_body(i):
            sv = sq_v[pl.ds(i * 16, 16)]
            iv = idx_v[pl.ds(i * 16, 16)] + lane_off
            plsc.addupdate_scatter(acc_s, [iv], sv)
            plsc.addupdate_scatter(acc_c, [iv], ones)

        @plsc.parallel_loop(c0, c1, 1, unroll=2)
        def _red_body(c):
            s = zero
            n = zero
            for r in range(16):
                s = s + acc_s[pl.ds(r * _B + c * 16, 16)]
                n = n + acc_c[pl.ds(r * _B + c * 16, 16)]
            red_s[pl.ds(c * 16, 16)] = s
            red_c[pl.ds(c * 16, 16)] = n

        pltpu.sync_copy(red_s, psum_hbm.at[wid])
        pltpu.sync_copy(red_c, pcnt_hbm.at[wid])

    return _sc_segsum


def _tc_combine_body(ps_ref, pc_ref, k_ref, o_ref):
    s = jnp.sum(ps_ref[...], axis=0, keepdims=True)
    c = jnp.sum(pc_ref[...], axis=0, keepdims=True)
    l_ = k_ref[0, 0]
    s2 = k_ref[0, 1]
    ch20 = k_ref[0, 2]
    ha2 = k_ref[0, 3]
    half = jnp.float32(0.5)
    d = jnp.float32(3.0) * (c - jnp.float32(1.0))
    ds2 = d * s2
    cterm = d * l_ + half * (ds2 - d)
    o_ref[...] = ch20 * c + cterm * c + ha2 * s


def _tc_combine(psum, pcnt, consts):
    return pl.pallas_call(
        _tc_combine_body,
        out_shape=jax.ShapeDtypeStruct((1, _B), jnp.float32),
    )(psum, pcnt, consts)


def kernel(h, x, graph_indices):
    l_, s2, ch20, ha2 = _schedule_consts()
    consts = jnp.stack([l_, s2, ch20, ha2]).reshape(1, 4)
    sq = _tc_sq(h, x)
    psum, pcnt = _get_sc_segsum()(sq, graph_indices)
    return _tc_combine(psum, pcnt, consts).reshape(_B)


# speedup vs baseline: 1.0207x; 1.0207x over previous
import functools

import numpy as np
import jax
import jax.numpy as jnp
from jax import lax
from jax.experimental import pallas as pl
from jax.experimental.pallas import tpu as pltpu
from jax.experimental.pallas import tpu_sc as plsc

_B = 1024
_N = 262144
_NODE_DIM = 20
_X_DIM = 3


def _schedule_consts():
    start = jnp.arccos(jnp.float32(0.95))
    end = jnp.arccos(jnp.float32(0.02))
    angles = start + jnp.float32(1.0) * (end - start)
    alpha = jnp.cos(angles)
    sigma = jnp.sin(angles)
    L = jnp.log(jnp.float32(1.0) / sigma)
    s2 = sigma * sigma
    half = jnp.float32(0.5)
    ch20 = jnp.float32(20.0) * (L - half * (jnp.float32(1.0) - s2))
    ha2 = half * (alpha * alpha)
    return L, s2, ch20, ha2

_GRID1 = 4
_BNL = _N // _GRID1


def _tc_sq_body(h_ref, x_ref, o_ref):
    hv = h_ref[...]
    xv = x_ref[...]
    o_ref[...] = jnp.sum(hv * hv, axis=0) + jnp.sum(xv * xv, axis=0)


def _tc_sq(h, x):
    return pl.pallas_call(
        _tc_sq_body,
        grid=(_GRID1,),
        in_specs=[
            pl.BlockSpec((_NODE_DIM, _BNL), lambda i: (0, i)),
            pl.BlockSpec((_X_DIM, _BNL), lambda i: (0, i)),
        ],
        out_specs=pl.BlockSpec((_BNL,), lambda i: (i,)),
        out_shape=jax.ShapeDtypeStruct((_N,), jnp.float32),
    )(h.T, x.T)


_NW = 32
_CHUNK = _N // _NW
_NVEC = _CHUNK // 16


@functools.cache
def _get_sc_segsum():
    mesh = plsc.VectorSubcoreMesh(core_axis_name="c", subcore_axis_name="s")

    @functools.partial(
        pl.kernel,
        mesh=mesh,
        out_type=[
            jax.ShapeDtypeStruct((_NW, _B), jnp.float32),
            jax.ShapeDtypeStruct((_NW, _B), jnp.float32),
        ],
        scratch_types=[
            pltpu.VMEM((_CHUNK,), jnp.float32),
            pltpu.VMEM((_CHUNK,), jnp.int32),
            pltpu.VMEM((16 * _B,), jnp.float32),
            pltpu.VMEM((16 * _B,), jnp.float32),
            pltpu.VMEM((_B,), jnp.float32),
            pltpu.VMEM((_B,), jnp.float32),
        ],
        compiler_params=pltpu.CompilerParams(needs_layout_passes=False),
    )
    def _sc_segsum(sq_hbm, idx_hbm, psum_hbm, pcnt_hbm,
                   sq_v, idx_v, acc_s, acc_c, red_s, red_c):
        wid = lax.axis_index("s") * 2 + lax.axis_index("c")
        base = wid * _CHUNK
        pltpu.sync_copy(sq_hbm.at[pl.ds(base, _CHUNK)], sq_v)
        pltpu.sync_copy(idx_hbm.at[pl.ds(base, _CHUNK)], idx_v)

        zero = jnp.zeros((16,), jnp.float32)
        ones = jnp.ones((16,), jnp.float32)
        lane_off = lax.iota(jnp.int32, 16) * _B

        lo = lax.reduce_min(idx_v[pl.ds(0, 16)], axes=(0,))
        hi = lax.reduce_max(idx_v[pl.ds(_CHUNK - 16, 16)], axes=(0,))
        c0 = lo // 16
        c1 = hi // 16 + 1

        @plsc.parallel_loop(c0, c1, 1, unroll=2)
        def _zero_body(c):
            for r in range(16):
                acc_s[pl.ds(r * _B + c * 16, 16)] = zero
                acc_c[pl.ds(r * _B + c * 16, 16)] = zero

        @plsc.parallel_loop(0, _B // 16, 1, unroll=8)
        def _zero_red_body(c):
            red_s[pl.ds(c * 16, 16)] = zero
            red_c[pl.ds(c * 16, 16)] = zero

        @plsc.parallel_loop(0, _NVEC, 1, unroll=8)
        def _acc_body(i):
            sv = sq_v[pl.ds(i * 16, 16)]
            iv = idx_v[pl.ds(i * 16, 16)] + lane_off
            plsc.addupdate_scatter(acc_s, [iv], sv)
            plsc.addupdate_scatter(acc_c, [iv], ones)

        @plsc.parallel_loop(c0, c1, 1, unroll=2)
        def _red_body(c):
            s = zero
            n = zero
            for r in range(16):
                s = s + acc_s[pl.ds(r * _B + c * 16, 16)]
                n = n + acc_c[pl.ds(r * _B + c * 16, 16)]
            red_s[pl.ds(c * 16, 16)] = s
            red_c[pl.ds(c * 16, 16)] = n

        pltpu.sync_copy(red_s, psum_hbm.at[wid])
        pltpu.sync_copy(red_c, pcnt_hbm.at[wid])

    return _sc_segsum


def _tc_combine_body(ps_ref, pc_ref, k_ref, o_ref):
    s = jnp.sum(ps_ref[...], axis=0, keepdims=True)
    c = jnp.sum(pc_ref[...], axis=0, keepdims=True)
    l_ = k_ref[0, 0]
    s2 = k_ref[0, 1]
    ch20 = k_ref[0, 2]
    ha2 = k_ref[0, 3]
    half = jnp.float32(0.5)
    d = jnp.float32(3.0) * (c - jnp.float32(1.0))
    ds2 = d * s2
    cterm = d * l_ + half * (ds2 - d)
    o_ref[...] = ch20 * c + cterm * c + ha2 * s


def _tc_combine(psum, pcnt, consts):
    return pl.pallas_call(
        _tc_combine_body,
        out_shape=jax.ShapeDtypeStruct((1, _B), jnp.float32),
    )(psum, pcnt, consts)


def kernel(h, x, graph_indices):
    l_, s2, ch20, ha2 = _schedule_consts()
    consts = jnp.stack([l_, s2, ch20, ha2]).reshape(1, 4)
    sq = _tc_sq(h, x)
    psum, pcnt = _get_sc_segsum()(sq, graph_indices)
    return _tc_combine(psum, pcnt, consts).reshape(_B)
